# baseline (speedup 1.0000x reference)
import jax
import jax.numpy as jnp
from jax import lax
from jax.experimental import pallas as pl
from jax.experimental.pallas import tpu as pltpu
from jax.experimental.pallas import tpu_sc as plsc

N = 10000
D = 128
E = 320000

NC = 2
NS = 16
NW = NC * NS

DCHUNK = 128
DROWS = 80
PCHUNK = 128
PROWS = 80
CHUNK = 32
NSUB = PROWS * (PCHUNK // CHUNK)
E_PAD = NW * PROWS * PCHUNK
N_ACC = 10240
INIT_PT = N_ACC // NS

_MESH = plsc.VectorSubcoreMesh(
    core_axis_name="c", subcore_axis_name="s", num_cores=NC, num_subcores=NS
)


def _deg_body(dst_hbm, ones_hbm, zeros_hbm, out_hbm, dst_v, ones_v, acc):
    c = lax.axis_index("c")
    s = lax.axis_index("s")
    wid = c * NS + s
    pltpu.sync_copy(zeros_hbm, acc.at[pl.ds(s * INIT_PT, INIT_PT)])
    pltpu.sync_copy(dst_hbm.at[pl.ds(wid * DROWS, DROWS)], dst_v)
    pltpu.sync_copy(ones_hbm, ones_v)
    plsc.subcore_barrier()

    def body(j, carry):
        pltpu.sync_copy(ones_v, acc.at[dst_v.at[j]], add=True)
        return carry

    lax.fori_loop(0, DROWS, body, 0)
    plsc.subcore_barrier()
    pltpu.sync_copy(
        acc.at[pl.ds(s * INIT_PT, INIT_PT)], out_hbm.at[c, pl.ds(s * INIT_PT, INIT_PT)]
    )


def _make_deg_kernel(interpret=False):
    return pl.kernel(
        _deg_body,
        out_type=jax.ShapeDtypeStruct((NC, N_ACC, D), jnp.float32),
        mesh=_MESH,
        scratch_types=[
            pltpu.VMEM((DROWS, DCHUNK), jnp.int32),
            pltpu.VMEM((DCHUNK, D), jnp.float32),
            pltpu.VMEM_SHARED((N_ACC, D), jnp.float32),
        ],
        interpret=interpret,
    )


def _fill(ref, nrows, value):
    vec = jnp.full((16,), value, jnp.float32)

    def row(r, carry):
        for k in range(D // 16):
            ref[r, pl.ds(k * 16, 16)] = vec
        return carry

    lax.fori_loop(0, nrows, row, 0)


_NBUF = 8
_DELAY = 2
_NGRP = NSUB // _NBUF
_SPR = PCHUNK // CHUNK


def _prop_body(g_hbm, packed_hbm, out_hbm, packed_v, src_st, dst_st, *scratch):
    rows = scratch[:_NBUF]
    acc = scratch[_NBUF]
    gsem = scratch[_NBUF + 1 : 2 * _NBUF + 1]
    ssem = scratch[2 * _NBUF + 1 :]
    c = lax.axis_index("c")
    s = lax.axis_index("s")
    wid = c * NS + s
    pltpu.sync_copy(packed_hbm.at[pl.ds(wid * PROWS, PROWS)], packed_v)
    _fill(rows[0], CHUNK, 0.0)
    for p in range(INIT_PT // CHUNK):
        pltpu.sync_copy(rows[0], acc.at[pl.ds(s * INIT_PT + p * CHUNK, CHUNK)])
    plsc.subcore_barrier()

    def unpack(jd, u, b):
        row = jd // _SPR if isinstance(jd, int) else lax.div(jd, _SPR)
        for k in range(CHUNK // 16):
            col = (u % _SPR) * CHUNK + k * 16
            v = packed_v[row, pl.ds(col, 16)]
            src_st[b, pl.ds(k * 16, 16)] = jnp.bitwise_and(v, 0xFFFF)
            dst_st[b, pl.ds(k * 16, 16)] = jnp.right_shift(v, 16)

    def wait_scatter(b):
        pltpu.make_async_copy(rows[b], acc.at[dst_st.at[b]], ssem[b]).wait()

    def wait_gather(b):
        pltpu.make_async_copy(g_hbm.at[src_st.at[b]], rows[b], gsem[b]).wait()

    def issue_gather(b):
        pltpu.async_copy(g_hbm.at[src_st.at[b]], rows[b], gsem[b])

    def issue_scatter(b):
        pltpu.async_copy(rows[b], acc.at[dst_st.at[b]], ssem[b], add=True)

    for b in range(_NBUF - _DELAY):
        unpack(b, b, b)
        issue_gather(b)

    for u in range(_NBUF):
        if u < _DELAY:
            unpack(u + _NBUF - _DELAY, u + _NBUF - _DELAY, u + _NBUF - _DELAY)
            issue_gather(u + _NBUF - _DELAY)
        else:
            bp = u - _DELAY
            wait_scatter(bp)
            unpack(u - _DELAY + _NBUF, u - _DELAY, bp)
            issue_gather(bp)
        wait_gather(u)
        issue_scatter(u)

    def group(grp, carry):
        for u in range(_NBUF):
            j = grp * _NBUF + u
            bp = (u - _DELAY) % _NBUF
            wait_scatter(bp)
            unpack(j - _DELAY + _NBUF, u - _DELAY, bp)
            issue_gather(bp)
            wait_gather(u)
            issue_scatter(u)
        return carry

    lax.fori_loop(1, _NGRP - 1, group, 0)

    j0 = NSUB - _NBUF
    for u in range(_NBUF):
        j = j0 + u
        bp = (u - _DELAY) % _NBUF
        wait_scatter(bp)
        if j - _DELAY + _NBUF < NSUB:
            unpack(j - _DELAY + _NBUF, u - _DELAY, bp)
            issue_gather(bp)
        wait_gather(u)
        issue_scatter(u)

    for i in range(_DELAY):
        b = (NSUB - _DELAY + i) % _NBUF
        pltpu.make_async_copy(rows[b], acc.at[dst_st.at[b]], ssem[b]).wait()
    plsc.subcore_barrier()
    pltpu.sync_copy(
        acc.at[pl.ds(s * INIT_PT, INIT_PT)], out_hbm.at[c, pl.ds(s * INIT_PT, INIT_PT)]
    )


def _make_prop_kernel(interpret=False):
    return pl.kernel(
        _prop_body,
        out_type=jax.ShapeDtypeStruct((NC, N_ACC, D), jnp.float32),
        mesh=_MESH,
        scratch_types=[
            pltpu.VMEM((PROWS, PCHUNK), jnp.int32),
            pltpu.VMEM((_NBUF, CHUNK), jnp.int32),
            pltpu.VMEM((_NBUF, CHUNK), jnp.int32),
        ]
        + [pltpu.VMEM((CHUNK, D), jnp.float32)] * _NBUF
        + [pltpu.VMEM_SHARED((N_ACC, D), jnp.float32)]
        + [pltpu.SemaphoreType.DMA] * (2 * _NBUF),
        interpret=interpret,
    )


_deg_kernel = _make_deg_kernel()
_prop_kernel = _make_prop_kernel()


_R = 1000


def _k1_body(x_ref, degp_ref, w_ref, g_ref, dinv_ref):
    deg = degp_ref[0, :, :1] + degp_ref[1, :, :1] + 1.0
    dinv = lax.rsqrt(deg)
    h = jnp.dot(x_ref[...], w_ref[...], preferred_element_type=jnp.float32)
    g_ref[...] = h * dinv
    dinv_ref[...] = dinv


def _k1_call(x, degp, W1):
    grid = (N // _R,)
    return pl.pallas_call(
        _k1_body,
        grid=grid,
        in_specs=[
            pl.BlockSpec((_R, D), lambda i: (i, 0)),
            pl.BlockSpec((NC, _R, D), lambda i: (0, i, 0)),
            pl.BlockSpec((D, D), lambda i: (0, 0)),
        ],
        out_specs=[
            pl.BlockSpec((_R, D), lambda i: (i, 0)),
            pl.BlockSpec((_R, 1), lambda i: (i, 0)),
        ],
        out_shape=[
            jax.ShapeDtypeStruct((N, D), jnp.float32),
            jax.ShapeDtypeStruct((N, 1), jnp.float32),
        ],
    )(x, degp, W1)


def _k2_body(a_ref, g_ref, dinv_ref, b_ref, w_ref, out_ref):
    dinv = dinv_ref[...]
    z = (a_ref[0] + a_ref[1] + g_ref[...]) * dinv + b_ref[...]
    z = jnp.maximum(z, 0.0)
    out_ref[...] = jnp.dot(z, w_ref[...], preferred_element_type=jnp.float32) * dinv


def _k2_call(A, g1, dinv, b1, W2):
    grid = (N // _R,)
    return pl.pallas_call(
        _k2_body,
        grid=grid,
        in_specs=[
            pl.BlockSpec((NC, _R, D), lambda i: (0, i, 0)),
            pl.BlockSpec((_R, D), lambda i: (i, 0)),
            pl.BlockSpec((_R, 1), lambda i: (i, 0)),
            pl.BlockSpec((1, D), lambda i: (0, 0)),
            pl.BlockSpec((D, D), lambda i: (0, 0)),
        ],
        out_specs=pl.BlockSpec((_R, D), lambda i: (i, 0)),
        out_shape=jax.ShapeDtypeStruct((N, D), jnp.float32),
    )(A, g1, dinv, b1, W2)


def _k3_body(a_ref, g_ref, dinv_ref, b_ref, out_ref):
    out_ref[...] = (a_ref[0] + a_ref[1] + g_ref[...]) * dinv_ref[...] + b_ref[...]


def _k3_call(A, g2, dinv, b2):
    grid = (N // _R,)
    return pl.pallas_call(
        _k3_body,
        grid=grid,
        in_specs=[
            pl.BlockSpec((NC, _R, D), lambda i: (0, i, 0)),
            pl.BlockSpec((_R, D), lambda i: (i, 0)),
            pl.BlockSpec((_R, 1), lambda i: (i, 0)),
            pl.BlockSpec((1, D), lambda i: (0, 0)),
        ],
        out_specs=pl.BlockSpec((_R, D), lambda i: (i, 0)),
        out_shape=jax.ShapeDtypeStruct((N, D), jnp.float32),
    )(A, g2, dinv, b2)


def kernel(x, edge_index, W1, b1, W2, b2):
    src = edge_index[0].astype(jnp.int32)
    dst = edge_index[1].astype(jnp.int32)
    pad = E_PAD - E
    src_f = jnp.concatenate([src, jnp.zeros((pad,), jnp.int32)])
    dst_f = jnp.concatenate([dst, jnp.full((pad,), N, jnp.int32)])
    dst_d = dst_f.reshape(-1, DCHUNK)
    packed = jnp.bitwise_or(src_f, jnp.left_shift(dst_f, 16)).reshape(-1, PCHUNK)

    ones128 = jnp.ones((DCHUNK, D), jnp.float32)
    zeros128 = jnp.zeros((INIT_PT, D), jnp.float32)
    degp = _deg_kernel(dst_d, ones128, zeros128)
    g1, dinv = _k1_call(x, degp, W1)
    A1 = _prop_kernel(g1, packed)
    g2 = _k2_call(A1, g1, dinv, jnp.reshape(b1, (1, D)), W2)
    A2 = _prop_kernel(g2, packed)
    out = _k3_call(A2, g2, dinv, jnp.reshape(b2, (1, D)))
    return out


# speedup vs baseline: 1.0154x; 1.0154x over previous
import jax
import jax.numpy as jnp
from jax import lax
from jax.experimental import pallas as pl
from jax.experimental.pallas import tpu as pltpu
from jax.experimental.pallas import tpu_sc as plsc

N = 10000
D = 128
E = 320000

NC = 2
NS = 16
NW = NC * NS

DCHUNK = 128
DROWS = 80
PCHUNK = 128
PROWS = 80
CHUNK = 32
NSUB = PROWS * (PCHUNK // CHUNK)
E_PAD = NW * PROWS * PCHUNK
N_ACC = 10240
INIT_PT = N_ACC // NS

_MESH = plsc.VectorSubcoreMesh(
    core_axis_name="c", subcore_axis_name="s", num_cores=NC, num_subcores=NS
)


def _deg_body(dst_hbm, ones_hbm, zeros_hbm, out_hbm, dst_v, ones_v, acc):
    c = lax.axis_index("c")
    s = lax.axis_index("s")
    wid = c * NS + s
    pltpu.sync_copy(zeros_hbm, acc.at[pl.ds(s * INIT_PT, INIT_PT)])
    pltpu.sync_copy(dst_hbm.at[pl.ds(wid * DROWS, DROWS)], dst_v)
    pltpu.sync_copy(ones_hbm, ones_v)
    plsc.subcore_barrier()

    def body(j, carry):
        pltpu.sync_copy(ones_v, acc.at[dst_v.at[j]], add=True)
        return carry

    lax.fori_loop(0, DROWS, body, 0)
    plsc.subcore_barrier()
    pltpu.sync_copy(
        acc.at[pl.ds(s * INIT_PT, INIT_PT)], out_hbm.at[c, pl.ds(s * INIT_PT, INIT_PT)]
    )


def _make_deg_kernel(interpret=False):
    return pl.kernel(
        _deg_body,
        out_type=jax.ShapeDtypeStruct((NC, N_ACC, D), jnp.float32),
        mesh=_MESH,
        scratch_types=[
            pltpu.VMEM((DROWS, DCHUNK), jnp.int32),
            pltpu.VMEM((DCHUNK, D), jnp.float32),
            pltpu.VMEM_SHARED((N_ACC, D), jnp.float32),
        ],
        interpret=interpret,
    )


def _fill(ref, nrows, value):
    vec = jnp.full((16,), value, jnp.float32)

    def row(r, carry):
        for k in range(D // 16):
            ref[r, pl.ds(k * 16, 16)] = vec
        return carry

    lax.fori_loop(0, nrows, row, 0)


_NBUF = 8
_DELAY = 2
_NGRP = NSUB // _NBUF
_SPR = PCHUNK // CHUNK


def _prop_body(g_hbm, packed_hbm, out_hbm, packed_v, src_st, dst_st, *scratch):
    rows = scratch[:_NBUF]
    acc = scratch[_NBUF]
    gsem = scratch[_NBUF + 1 : 2 * _NBUF + 1]
    ssem = scratch[2 * _NBUF + 1 :]
    c = lax.axis_index("c")
    s = lax.axis_index("s")
    wid = c * NS + s
    pltpu.sync_copy(packed_hbm.at[pl.ds(wid * PROWS, PROWS)], packed_v)
    _fill(rows[0], CHUNK, 0.0)
    for p in range(INIT_PT // CHUNK):
        pltpu.sync_copy(rows[0], acc.at[pl.ds(s * INIT_PT + p * CHUNK, CHUNK)])
    plsc.subcore_barrier()

    def unpack(jd, u, b):
        row = jd // _SPR if isinstance(jd, int) else lax.div(jd, _SPR)
        for k in range(CHUNK // 16):
            col = (u % _SPR) * CHUNK + k * 16
            v = packed_v[row, pl.ds(col, 16)]
            src_st[b, pl.ds(k * 16, 16)] = jnp.bitwise_and(v, 0xFFFF)
            dst_st[b, pl.ds(k * 16, 16)] = jnp.right_shift(v, 16)

    def wait_scatter(b):
        pltpu.make_async_copy(rows[b], acc.at[dst_st.at[b]], ssem[b]).wait()

    def wait_gather(b):
        pltpu.make_async_copy(g_hbm.at[src_st.at[b]], rows[b], gsem[b]).wait()

    def issue_gather(b):
        pltpu.async_copy(g_hbm.at[src_st.at[b]], rows[b], gsem[b])

    def issue_scatter(b):
        pltpu.async_copy(rows[b], acc.at[dst_st.at[b]], ssem[b], add=True)

    for b in range(_NBUF - _DELAY):
        unpack(b, b, b)
        issue_gather(b)

    for u in range(_NBUF):
        if u < _DELAY:
            unpack(u + _NBUF - _DELAY, u + _NBUF - _DELAY, u + _NBUF - _DELAY)
            issue_gather(u + _NBUF - _DELAY)
        else:
            bp = u - _DELAY
            wait_scatter(bp)
            unpack(u - _DELAY + _NBUF, u - _DELAY, bp)
            issue_gather(bp)
        wait_gather(u)
        issue_scatter(u)

    def group(grp, carry):
        for u in range(_NBUF):
            j = grp * _NBUF + u
            bp = (u - _DELAY) % _NBUF
            wait_scatter(bp)
            unpack(j - _DELAY + _NBUF, u - _DELAY, bp)
            issue_gather(bp)
            wait_gather(u)
            issue_scatter(u)
        return carry

    lax.fori_loop(1, _NGRP - 1, group, 0)

    j0 = NSUB - _NBUF
    for u in range(_NBUF):
        j = j0 + u
        bp = (u - _DELAY) % _NBUF
        wait_scatter(bp)
        if j - _DELAY + _NBUF < NSUB:
            unpack(j - _DELAY + _NBUF, u - _DELAY, bp)
            issue_gather(bp)
        wait_gather(u)
        issue_scatter(u)

    for i in range(_DELAY):
        b = (NSUB - _DELAY + i) % _NBUF
        pltpu.make_async_copy(rows[b], acc.at[dst_st.at[b]], ssem[b]).wait()
    plsc.subcore_barrier()
    pltpu.sync_copy(
        acc.at[pl.ds(s * INIT_PT, INIT_PT)], out_hbm.at[c, pl.ds(s * INIT_PT, INIT_PT)]
    )


def _make_prop_kernel(interpret=False):
    return pl.kernel(
        _prop_body,
        out_type=jax.ShapeDtypeStruct((NC, N_ACC, D), jnp.float32),
        mesh=_MESH,
        scratch_types=[
            pltpu.VMEM((PROWS, PCHUNK), jnp.int32),
            pltpu.VMEM((_NBUF, CHUNK), jnp.int32),
            pltpu.VMEM((_NBUF, CHUNK), jnp.int32),
        ]
        + [pltpu.VMEM((CHUNK, D), jnp.float32)] * _NBUF
        + [pltpu.VMEM_SHARED((N_ACC, D), jnp.float32)]
        + [pltpu.SemaphoreType.DMA] * (2 * _NBUF),
        interpret=interpret,
    )


_deg_kernel = _make_deg_kernel()
_prop_kernel = _make_prop_kernel()


_R = 1000


def _k1_body(x_ref, degp_ref, w_ref, g_ref, dinv_ref):
    deg = degp_ref[0, :, :1] + degp_ref[1, :, :1] + 1.0
    dinv = lax.rsqrt(deg)
    h = jnp.dot(x_ref[...], w_ref[...], preferred_element_type=jnp.float32)
    g_ref[...] = h * dinv
    dinv_ref[...] = dinv


def _k1_call(x, degp, W1):
    grid = (N // _R,)
    return pl.pallas_call(
        _k1_body,
        grid=grid,
        in_specs=[
            pl.BlockSpec((_R, D), lambda i: (i, 0)),
            pl.BlockSpec((NC, _R, D), lambda i: (0, i, 0)),
            pl.BlockSpec((D, D), lambda i: (0, 0)),
        ],
        out_specs=[
            pl.BlockSpec((_R, D), lambda i: (i, 0)),
            pl.BlockSpec((_R, 1), lambda i: (i, 0)),
        ],
        out_shape=[
            jax.ShapeDtypeStruct((N, D), jnp.float32),
            jax.ShapeDtypeStruct((N, 1), jnp.float32),
        ],
    )(x, degp, W1)


def _k2_body(a_ref, g_ref, dinv_ref, b_ref, w_ref, out_ref):
    dinv = dinv_ref[...]
    z = (a_ref[0] + a_ref[1] + g_ref[...]) * dinv + b_ref[...]
    z = jnp.maximum(z, 0.0)
    out_ref[...] = jnp.dot(z, w_ref[...], preferred_element_type=jnp.float32) * dinv


def _k2_call(A, g1, dinv, b1, W2):
    grid = (N // _R,)
    return pl.pallas_call(
        _k2_body,
        grid=grid,
        in_specs=[
            pl.BlockSpec((NC, _R, D), lambda i: (0, i, 0)),
            pl.BlockSpec((_R, D), lambda i: (i, 0)),
            pl.BlockSpec((_R, 1), lambda i: (i, 0)),
            pl.BlockSpec((1, D), lambda i: (0, 0)),
            pl.BlockSpec((D, D), lambda i: (0, 0)),
        ],
        out_specs=pl.BlockSpec((_R, D), lambda i: (i, 0)),
        out_shape=jax.ShapeDtypeStruct((N, D), jnp.float32),
    )(A, g1, dinv, b1, W2)


def _k3_body(a_ref, g_ref, dinv_ref, b_ref, out_ref):
    out_ref[...] = (a_ref[0] + a_ref[1] + g_ref[...]) * dinv_ref[...] + b_ref[...]


def _k3_call(A, g2, dinv, b2):
    grid = (N // _R,)
    return pl.pallas_call(
        _k3_body,
        grid=grid,
        in_specs=[
            pl.BlockSpec((NC, _R, D), lambda i: (0, i, 0)),
            pl.BlockSpec((_R, D), lambda i: (i, 0)),
            pl.BlockSpec((_R, 1), lambda i: (i, 0)),
            pl.BlockSpec((1, D), lambda i: (0, 0)),
        ],
        out_specs=pl.BlockSpec((_R, D), lambda i: (i, 0)),
        out_shape=jax.ShapeDtypeStruct((N, D), jnp.float32),
    )(A, g2, dinv, b2)


def kernel(x, edge_index, W1, b1, W2, b2):
    src = edge_index[0].astype(jnp.int32)
    dst = edge_index[1].astype(jnp.int32)
    pad = E_PAD - E
    src_f = jnp.concatenate([src, jnp.zeros((pad,), jnp.int32)])
    pad_dst = N + jnp.arange(pad, dtype=jnp.int32) % (N_ACC - N)
    dst_f = jnp.concatenate([dst, pad_dst])
    dst_d = dst_f.reshape(-1, DCHUNK)
    packed = jnp.bitwise_or(src_f, jnp.left_shift(dst_f, 16)).reshape(-1, PCHUNK)

    ones128 = jnp.ones((DCHUNK, D), jnp.float32)
    zeros128 = jnp.zeros((INIT_PT, D), jnp.float32)
    degp = _deg_kernel(dst_d, ones128, zeros128)
    g1, dinv = _k1_call(x, degp, W1)
    A1 = _prop_kernel(g1, packed)
    g2 = _k2_call(A1, g1, dinv, jnp.reshape(b1, (1, D)), W2)
    A2 = _prop_kernel(g2, packed)
    out = _k3_call(A2, g2, dinv, jnp.reshape(b2, (1, D)))
    return out


# speedup vs baseline: 3.2773x; 3.2277x over previous
import jax
import jax.numpy as jnp
from jax import lax
from jax.experimental import pallas as pl
from jax.experimental.pallas import tpu as pltpu
from jax.experimental.pallas import tpu_sc as plsc

N = 10000
D = 128
E = 320000

NC = 2
NS = 16
NW = NC * NS

DCHUNK = 128
DROWS = 80
PCHUNK = 128
PROWS = 80
CHUNK = 32
NSUB = PROWS * (PCHUNK // CHUNK)
E_PAD = NW * PROWS * PCHUNK
N_ACC = 10240
INIT_PT = N_ACC // NS

_MESH = plsc.VectorSubcoreMesh(
    core_axis_name="c", subcore_axis_name="s", num_cores=NC, num_subcores=NS
)


def _deg_body(dst_hbm, ones_hbm, zeros_hbm, out_hbm, dst_v, ones_v, acc):
    c = lax.axis_index("c")
    s = lax.axis_index("s")
    wid = c * NS + s
    pltpu.sync_copy(zeros_hbm, acc.at[pl.ds(s * INIT_PT, INIT_PT)])
    pltpu.sync_copy(dst_hbm.at[pl.ds(wid * DROWS, DROWS)], dst_v)
    pltpu.sync_copy(ones_hbm, ones_v)
    plsc.subcore_barrier()

    def body(j, carry):
        pltpu.sync_copy(ones_v, acc.at[dst_v.at[j]], add=True)
        return carry

    lax.fori_loop(0, DROWS, body, 0)
    plsc.subcore_barrier()
    pltpu.sync_copy(
        acc.at[pl.ds(s * INIT_PT, INIT_PT)], out_hbm.at[c, pl.ds(s * INIT_PT, INIT_PT)]
    )


def _make_deg_kernel(interpret=False):
    return pl.kernel(
        _deg_body,
        out_type=jax.ShapeDtypeStruct((NC, N_ACC, D), jnp.float32),
        mesh=_MESH,
        scratch_types=[
            pltpu.VMEM((DROWS, DCHUNK), jnp.int32),
            pltpu.VMEM((DCHUNK, D), jnp.float32),
            pltpu.VMEM_SHARED((N_ACC, D), jnp.float32),
        ],
        interpret=interpret,
    )


def _fill(ref, nrows, value):
    vec = jnp.full((16,), value, jnp.float32)

    def row(r, carry):
        for k in range(D // 16):
            ref[r, pl.ds(k * 16, 16)] = vec
        return carry

    lax.fori_loop(0, nrows, row, 0)


_NBUF = 8
_DELAY = 2
_NGRP = NSUB // _NBUF
_SPR = PCHUNK // CHUNK


def _prop_body(g_hbm, packed_hbm, out_hbm, packed_v, src_st, dst_st, *scratch):
    rows = scratch[:_NBUF]
    acc = scratch[_NBUF]
    gsem = scratch[_NBUF + 1 : 2 * _NBUF + 1]
    ssem = scratch[2 * _NBUF + 1 :]
    c = lax.axis_index("c")
    s = lax.axis_index("s")
    wid = c * NS + s
    pltpu.sync_copy(packed_hbm.at[pl.ds(wid * PROWS, PROWS)], packed_v)
    _fill(rows[0], CHUNK, 0.0)
    for p in range(INIT_PT // CHUNK):
        pltpu.sync_copy(rows[0], acc.at[pl.ds(s * INIT_PT + p * CHUNK, CHUNK)])
    plsc.subcore_barrier()

    def unpack(jd, u, b):
        row = jd // _SPR if isinstance(jd, int) else lax.div(jd, _SPR)
        for k in range(CHUNK // 16):
            col = (u % _SPR) * CHUNK + k * 16
            v = packed_v[row, pl.ds(col, 16)]
            src_st[b, pl.ds(k * 16, 16)] = jnp.bitwise_and(v, 0xFFFF)
            dst_st[b, pl.ds(k * 16, 16)] = jnp.right_shift(v, 16)

    def wait_scatter(b):
        pltpu.make_async_copy(rows[b], acc.at[dst_st.at[b]], ssem[b]).wait()

    def wait_gather(b):
        pltpu.make_async_copy(g_hbm.at[src_st.at[b]], rows[b], gsem[b]).wait()

    def issue_gather(b):
        pltpu.async_copy(g_hbm.at[src_st.at[b]], rows[b], gsem[b])

    def issue_scatter(b):
        pltpu.async_copy(rows[b], acc.at[dst_st.at[b]], ssem[b], add=True)

    for b in range(_NBUF - _DELAY):
        unpack(b, b, b)
        issue_gather(b)

    for u in range(_NBUF):
        if u < _DELAY:
            unpack(u + _NBUF - _DELAY, u + _NBUF - _DELAY, u + _NBUF - _DELAY)
            issue_gather(u + _NBUF - _DELAY)
        else:
            bp = u - _DELAY
            wait_scatter(bp)
            unpack(u - _DELAY + _NBUF, u - _DELAY, bp)
            issue_gather(bp)
        wait_gather(u)
        issue_scatter(u)

    def group(grp, carry):
        for u in range(_NBUF):
            j = grp * _NBUF + u
            bp = (u - _DELAY) % _NBUF
            wait_scatter(bp)
            unpack(j - _DELAY + _NBUF, u - _DELAY, bp)
            issue_gather(bp)
            wait_gather(u)
            issue_scatter(u)
        return carry

    lax.fori_loop(1, _NGRP - 1, group, 0)

    j0 = NSUB - _NBUF
    for u in range(_NBUF):
        j = j0 + u
        bp = (u - _DELAY) % _NBUF
        wait_scatter(bp)
        if j - _DELAY + _NBUF < NSUB:
            unpack(j - _DELAY + _NBUF, u - _DELAY, bp)
            issue_gather(bp)
        wait_gather(u)
        issue_scatter(u)

    for i in range(_DELAY):
        b = (NSUB - _DELAY + i) % _NBUF
        pltpu.make_async_copy(rows[b], acc.at[dst_st.at[b]], ssem[b]).wait()
    plsc.subcore_barrier()
    pltpu.sync_copy(
        acc.at[pl.ds(s * INIT_PT, INIT_PT)], out_hbm.at[c, pl.ds(s * INIT_PT, INIT_PT)]
    )


def _make_prop_kernel(interpret=False):
    return pl.kernel(
        _prop_body,
        out_type=jax.ShapeDtypeStruct((NC, N_ACC, D), jnp.float32),
        mesh=_MESH,
        scratch_types=[
            pltpu.VMEM((PROWS, PCHUNK), jnp.int32),
            pltpu.VMEM((_NBUF, CHUNK), jnp.int32),
            pltpu.VMEM((_NBUF, CHUNK), jnp.int32),
        ]
        + [pltpu.VMEM((CHUNK, D), jnp.float32)] * _NBUF
        + [pltpu.VMEM_SHARED((N_ACC, D), jnp.float32)]
        + [pltpu.SemaphoreType.DMA] * (2 * _NBUF),
        interpret=interpret,
    )


_deg_kernel = _make_deg_kernel()
_prop_kernel = _make_prop_kernel()


_R = 1000


def _k1_body(x_ref, degp_ref, w_ref, g_ref, dinv_ref):
    deg = degp_ref[0, :, :1] + degp_ref[1, :, :1] + 1.0
    dinv = lax.rsqrt(deg)
    h = jnp.dot(x_ref[...], w_ref[...], preferred_element_type=jnp.float32)
    g_ref[...] = h * dinv
    dinv_ref[...] = dinv


def _k1_call(x, degp, W1):
    grid = (N // _R,)
    return pl.pallas_call(
        _k1_body,
        grid=grid,
        in_specs=[
            pl.BlockSpec((_R, D), lambda i: (i, 0)),
            pl.BlockSpec((NC, _R, D), lambda i: (0, i, 0)),
            pl.BlockSpec((D, D), lambda i: (0, 0)),
        ],
        out_specs=[
            pl.BlockSpec((_R, D), lambda i: (i, 0)),
            pl.BlockSpec((_R, 1), lambda i: (i, 0)),
        ],
        out_shape=[
            jax.ShapeDtypeStruct((N, D), jnp.float32),
            jax.ShapeDtypeStruct((N, 1), jnp.float32),
        ],
    )(x, degp, W1)


def _k2_body(a_ref, g_ref, dinv_ref, b_ref, w_ref, out_ref):
    dinv = dinv_ref[...]
    z = (a_ref[0] + a_ref[1] + g_ref[...]) * dinv + b_ref[...]
    z = jnp.maximum(z, 0.0)
    out_ref[...] = jnp.dot(z, w_ref[...], preferred_element_type=jnp.float32) * dinv


def _k2_call(A, g1, dinv, b1, W2):
    grid = (N // _R,)
    return pl.pallas_call(
        _k2_body,
        grid=grid,
        in_specs=[
            pl.BlockSpec((NC, _R, D), lambda i: (0, i, 0)),
            pl.BlockSpec((_R, D), lambda i: (i, 0)),
            pl.BlockSpec((_R, 1), lambda i: (i, 0)),
            pl.BlockSpec((1, D), lambda i: (0, 0)),
            pl.BlockSpec((D, D), lambda i: (0, 0)),
        ],
        out_specs=pl.BlockSpec((_R, D), lambda i: (i, 0)),
        out_shape=jax.ShapeDtypeStruct((N, D), jnp.float32),
    )(A, g1, dinv, b1, W2)


def _k3_body(a_ref, g_ref, dinv_ref, b_ref, out_ref):
    out_ref[...] = (a_ref[0] + a_ref[1] + g_ref[...]) * dinv_ref[...] + b_ref[...]


def _k3_call(A, g2, dinv, b2):
    grid = (N // _R,)
    return pl.pallas_call(
        _k3_body,
        grid=grid,
        in_specs=[
            pl.BlockSpec((NC, _R, D), lambda i: (0, i, 0)),
            pl.BlockSpec((_R, D), lambda i: (i, 0)),
            pl.BlockSpec((_R, 1), lambda i: (i, 0)),
            pl.BlockSpec((1, D), lambda i: (0, 0)),
        ],
        out_specs=pl.BlockSpec((_R, D), lambda i: (i, 0)),
        out_shape=jax.ShapeDtypeStruct((N, D), jnp.float32),
    )(A, g2, dinv, b2)


def kernel(x, edge_index, W1, b1, W2, b2):
    src = edge_index[0].astype(jnp.int32)
    dst = edge_index[1].astype(jnp.int32)
    pad = E_PAD - E
    pad_src = jnp.arange(pad, dtype=jnp.int32) * 64 % N
    src_f = jnp.concatenate([src, pad_src])
    pad_dst = N + jnp.arange(pad, dtype=jnp.int32) % (N_ACC - N)
    dst_f = jnp.concatenate([dst, pad_dst])
    dst_d = dst_f.reshape(-1, DCHUNK)
    packed = jnp.bitwise_or(src_f, jnp.left_shift(dst_f, 16)).reshape(-1, PCHUNK)

    ones128 = jnp.ones((DCHUNK, D), jnp.float32)
    zeros128 = jnp.zeros((INIT_PT, D), jnp.float32)
    degp = _deg_kernel(dst_d, ones128, zeros128)
    g1, dinv = _k1_call(x, degp, W1)
    A1 = _prop_kernel(g1, packed)
    g2 = _k2_call(A1, g1, dinv, jnp.reshape(b1, (1, D)), W2)
    A2 = _prop_kernel(g2, packed)
    out = _k3_call(A2, g2, dinv, jnp.reshape(b2, (1, D)))
    return out


# speedup vs baseline: 3.2774x; 1.0000x over previous
import jax
import jax.numpy as jnp
from jax import lax
from jax.experimental import pallas as pl
from jax.experimental.pallas import tpu as pltpu
from jax.experimental.pallas import tpu_sc as plsc

N = 10000
D = 128
E = 320000

NC = 2
NS = 16
NW = NC * NS

DCHUNK = 128
DROWS = 80
PCHUNK = 128
PROWS = 80
CHUNK = 32
NSUB = PROWS * (PCHUNK // CHUNK)
E_PAD = NW * PROWS * PCHUNK
N_ACC = 10240
INIT_PT = N_ACC // NS

_MESH = plsc.VectorSubcoreMesh(
    core_axis_name="c", subcore_axis_name="s", num_cores=NC, num_subcores=NS
)


_DDEPTH = 4


def _deg_body(dst_hbm, ones_hbm, zeros_hbm, out_hbm, dst_v, ones_v, acc, sem):
    c = lax.axis_index("c")
    s = lax.axis_index("s")
    wid = c * NS + s
    pltpu.sync_copy(zeros_hbm, acc.at[pl.ds(s * INIT_PT, INIT_PT)])
    pltpu.sync_copy(dst_hbm.at[pl.ds(wid * DROWS, DROWS)], dst_v)
    pltpu.sync_copy(ones_hbm, ones_v)
    plsc.subcore_barrier()

    def body(j, carry):
        pltpu.sync_copy(ones_v, acc.at[dst_v.at[j]], add=True)
        return carry

    lax.fori_loop(0, DROWS, body, 0)
    plsc.subcore_barrier()
    pltpu.sync_copy(
        acc.at[pl.ds(s * INIT_PT, INIT_PT)], out_hbm.at[c, pl.ds(s * INIT_PT, INIT_PT)]
    )


def _make_deg_kernel(interpret=False):
    return pl.kernel(
        _deg_body,
        out_type=jax.ShapeDtypeStruct((NC, N_ACC, D), jnp.float32),
        mesh=_MESH,
        scratch_types=[
            pltpu.VMEM((DROWS, DCHUNK), jnp.int32),
            pltpu.VMEM((DCHUNK, D), jnp.float32),
            pltpu.VMEM_SHARED((N_ACC, D), jnp.float32),
            pltpu.SemaphoreType.DMA((_DDEPTH,)),
        ],
        interpret=interpret,
    )


def _fill(ref, nrows, value):
    vec = jnp.full((16,), value, jnp.float32)

    def row(r, carry):
        for k in range(D // 16):
            ref[r, pl.ds(k * 16, 16)] = vec
        return carry

    lax.fori_loop(0, nrows, row, 0)


_NBUF = 8
_DELAY = 2
_NGRP = NSUB // _NBUF
_SPR = PCHUNK // CHUNK


def _prop_body(g_hbm, packed_hbm, out_hbm, packed_v, src_st, dst_st, *scratch):
    rows = scratch[:_NBUF]
    acc = scratch[_NBUF]
    gsem = scratch[_NBUF + 1 : 2 * _NBUF + 1]
    ssem = scratch[2 * _NBUF + 1 :]
    c = lax.axis_index("c")
    s = lax.axis_index("s")
    wid = c * NS + s
    pltpu.sync_copy(packed_hbm.at[pl.ds(wid * PROWS, PROWS)], packed_v)
    _fill(rows[0], CHUNK, 0.0)
    for p in range(INIT_PT // CHUNK):
        pltpu.sync_copy(rows[0], acc.at[pl.ds(s * INIT_PT + p * CHUNK, CHUNK)])
    plsc.subcore_barrier()

    def unpack(jd, u, b):
        row = jd // _SPR if isinstance(jd, int) else lax.div(jd, _SPR)
        for k in range(CHUNK // 16):
            col = (u % _SPR) * CHUNK + k * 16
            v = packed_v[row, pl.ds(col, 16)]
            src_st[b, pl.ds(k * 16, 16)] = jnp.bitwise_and(v, 0xFFFF)
            dst_st[b, pl.ds(k * 16, 16)] = jnp.right_shift(v, 16)

    def wait_scatter(b):
        pltpu.make_async_copy(rows[b], acc.at[dst_st.at[b]], ssem[b]).wait()

    def wait_gather(b):
        pltpu.make_async_copy(g_hbm.at[src_st.at[b]], rows[b], gsem[b]).wait()

    def issue_gather(b):
        pltpu.async_copy(g_hbm.at[src_st.at[b]], rows[b], gsem[b])

    def issue_scatter(b):
        pltpu.async_copy(rows[b], acc.at[dst_st.at[b]], ssem[b], add=True)

    for b in range(_NBUF - _DELAY):
        unpack(b, b, b)
        issue_gather(b)

    for u in range(_NBUF):
        if u < _DELAY:
            unpack(u + _NBUF - _DELAY, u + _NBUF - _DELAY, u + _NBUF - _DELAY)
            issue_gather(u + _NBUF - _DELAY)
        else:
            bp = u - _DELAY
            wait_scatter(bp)
            unpack(u - _DELAY + _NBUF, u - _DELAY, bp)
            issue_gather(bp)
        wait_gather(u)
        issue_scatter(u)

    def group(grp, carry):
        for u in range(_NBUF):
            j = grp * _NBUF + u
            bp = (u - _DELAY) % _NBUF
            wait_scatter(bp)
            unpack(j - _DELAY + _NBUF, u - _DELAY, bp)
            issue_gather(bp)
            wait_gather(u)
            issue_scatter(u)
        return carry

    lax.fori_loop(1, _NGRP - 1, group, 0)

    j0 = NSUB - _NBUF
    for u in range(_NBUF):
        j = j0 + u
        bp = (u - _DELAY) % _NBUF
        wait_scatter(bp)
        if j - _DELAY + _NBUF < NSUB:
            unpack(j - _DELAY + _NBUF, u - _DELAY, bp)
            issue_gather(bp)
        wait_gather(u)
        issue_scatter(u)

    for i in range(_DELAY):
        b = (NSUB - _DELAY + i) % _NBUF
        pltpu.make_async_copy(rows[b], acc.at[dst_st.at[b]], ssem[b]).wait()
    plsc.subcore_barrier()
    pltpu.sync_copy(
        acc.at[pl.ds(s * INIT_PT, INIT_PT)], out_hbm.at[c, pl.ds(s * INIT_PT, INIT_PT)]
    )


def _make_prop_kernel(interpret=False):
    return pl.kernel(
        _prop_body,
        out_type=jax.ShapeDtypeStruct((NC, N_ACC, D), jnp.float32),
        mesh=_MESH,
        scratch_types=[
            pltpu.VMEM((PROWS, PCHUNK), jnp.int32),
            pltpu.VMEM((_NBUF, CHUNK), jnp.int32),
            pltpu.VMEM((_NBUF, CHUNK), jnp.int32),
        ]
        + [pltpu.VMEM((CHUNK, D), jnp.float32)] * _NBUF
        + [pltpu.VMEM_SHARED((N_ACC, D), jnp.float32)]
        + [pltpu.SemaphoreType.DMA] * (2 * _NBUF),
        interpret=interpret,
    )


_deg_kernel = _make_deg_kernel()
_prop_kernel = _make_prop_kernel()


_R = 1000


def _k1_body(x_ref, degp_ref, w_ref, g_ref, dinv_ref):
    deg = degp_ref[0, :, :1] + degp_ref[1, :, :1] + 1.0
    dinv = lax.rsqrt(deg)
    h = jnp.dot(x_ref[...], w_ref[...], preferred_element_type=jnp.float32)
    g_ref[...] = h * dinv
    dinv_ref[...] = dinv


def _k1_call(x, degp, W1):
    grid = (N // _R,)
    return pl.pallas_call(
        _k1_body,
        grid=grid,
        in_specs=[
            pl.BlockSpec((_R, D), lambda i: (i, 0)),
            pl.BlockSpec((NC, _R, D), lambda i: (0, i, 0)),
            pl.BlockSpec((D, D), lambda i: (0, 0)),
        ],
        out_specs=[
            pl.BlockSpec((_R, D), lambda i: (i, 0)),
            pl.BlockSpec((_R, 1), lambda i: (i, 0)),
        ],
        out_shape=[
            jax.ShapeDtypeStruct((N, D), jnp.float32),
            jax.ShapeDtypeStruct((N, 1), jnp.float32),
        ],
    )(x, degp, W1)


def _k2_body(a_ref, g_ref, dinv_ref, b_ref, w_ref, out_ref):
    dinv = dinv_ref[...]
    z = (a_ref[0] + a_ref[1] + g_ref[...]) * dinv + b_ref[...]
    z = jnp.maximum(z, 0.0)
    out_ref[...] = jnp.dot(z, w_ref[...], preferred_element_type=jnp.float32) * dinv


def _k2_call(A, g1, dinv, b1, W2):
    grid = (N // _R,)
    return pl.pallas_call(
        _k2_body,
        grid=grid,
        in_specs=[
            pl.BlockSpec((NC, _R, D), lambda i: (0, i, 0)),
            pl.BlockSpec((_R, D), lambda i: (i, 0)),
            pl.BlockSpec((_R, 1), lambda i: (i, 0)),
            pl.BlockSpec((1, D), lambda i: (0, 0)),
            pl.BlockSpec((D, D), lambda i: (0, 0)),
        ],
        out_specs=pl.BlockSpec((_R, D), lambda i: (i, 0)),
        out_shape=jax.ShapeDtypeStruct((N, D), jnp.float32),
    )(A, g1, dinv, b1, W2)


def _k3_body(a_ref, g_ref, dinv_ref, b_ref, out_ref):
    out_ref[...] = (a_ref[0] + a_ref[1] + g_ref[...]) * dinv_ref[...] + b_ref[...]


def _k3_call(A, g2, dinv, b2):
    grid = (N // _R,)
    return pl.pallas_call(
        _k3_body,
        grid=grid,
        in_specs=[
            pl.BlockSpec((NC, _R, D), lambda i: (0, i, 0)),
            pl.BlockSpec((_R, D), lambda i: (i, 0)),
            pl.BlockSpec((_R, 1), lambda i: (i, 0)),
            pl.BlockSpec((1, D), lambda i: (0, 0)),
        ],
        out_specs=pl.BlockSpec((_R, D), lambda i: (i, 0)),
        out_shape=jax.ShapeDtypeStruct((N, D), jnp.float32),
    )(A, g2, dinv, b2)


def kernel(x, edge_index, W1, b1, W2, b2):
    src = edge_index[0].astype(jnp.int32)
    dst = edge_index[1].astype(jnp.int32)
    pad = E_PAD - E
    pad_src = jnp.arange(pad, dtype=jnp.int32) * 64 % N
    src_f = jnp.concatenate([src, pad_src])
    pad_dst = N + jnp.arange(pad, dtype=jnp.int32) % (N_ACC - N)
    dst_f = jnp.concatenate([dst, pad_dst])
    dst_d = dst_f.reshape(-1, DCHUNK)
    packed = jnp.bitwise_or(src_f, jnp.left_shift(dst_f, 16)).reshape(-1, PCHUNK)

    ones128 = jnp.ones((DCHUNK, D), jnp.float32)
    zeros128 = jnp.zeros((INIT_PT, D), jnp.float32)
    degp = _deg_kernel(dst_d, ones128, zeros128)
    g1, dinv = _k1_call(x, degp, W1)
    A1 = _prop_kernel(g1, packed)
    g2 = _k2_call(A1, g1, dinv, jnp.reshape(b1, (1, D)), W2)
    A2 = _prop_kernel(g2, packed)
    out = _k3_call(A2, g2, dinv, jnp.reshape(b2, (1, D)))
    return out
